# baseline (speedup 1.0000x reference)
import jax
import jax.numpy as jnp
from jax import lax
from jax.experimental import pallas as pl
from jax.experimental.pallas import tpu as pltpu
from jax.experimental.pallas import tpu_sc as plsc

NUM_EMBEDDINGS = 1000000
EMBEDDING_DIM = 32
BATCH = 4096
SEQ_LEN = 200

_B = BATCH * SEQ_LEN
_NC = 2
_NS = 16
_NW = _NC * _NS
_PER_W = _B // _NW
_BPW = BATCH // _NW
_BPC = 2
_CHUNK = _BPC * SEQ_LEN
_NCHUNK = _PER_W // _CHUNK
_JT = SEQ_LEN // 8


def _body(xq_hbm, w_hbm, out_hbm, idxr_v, idx_v, rows_v, trans_v, gsems, wsems):
    wid = lax.axis_index("s") * _NC + lax.axis_index("c")
    iot = lax.iota(jnp.int32, 16)
    iot16 = iot + 16
    biot = iot * SEQ_LEN
    zero16 = jnp.full((16,), 0, jnp.int32)

    pltpu.sync_copy(xq_hbm.at[:, wid], idxr_v)

    @plsc.parallel_loop(0, SEQ_LEN, unroll=8)
    def _(q):
        jt = q >> 3
        j8 = q & 7
        for k in range(8):
            v = idxr_v[jt, j8, pl.ds(16 * k, 16)]
            plsc.store_scatter(idx_v, [biot + (16 * k * SEQ_LEN) + q], v)

    def start_gather(k, pb):
        pltpu.make_async_copy(
            w_hbm.at[idx_v.at[pl.ds(k * _CHUNK, _CHUNK)]],
            rows_v.at[pb],
            gsems.at[pb],
        ).start()

    def wait_gather(pb):
        pltpu.make_async_copy(
            w_hbm.at[idx_v.at[pl.ds(0, _CHUNK)]], rows_v.at[pb], gsems.at[pb]
        ).wait()

    def start_write(k, pb):
        b0 = wid * _BPW + k * _BPC
        for bi in range(_BPC):
            pltpu.make_async_copy(
                trans_v.at[pb, bi], out_hbm.at[b0 + bi], wsems.at[pb]
            ).start()

    def wait_write(pb):
        pltpu.make_async_copy(
            trans_v.at[pb], out_hbm.at[pl.ds(0, _BPC)], wsems.at[pb]
        ).wait()

    def transpose(pb):
        @plsc.parallel_loop(0, _CHUNK, unroll=8)
        def _(r):
            bi = jnp.where(r >= SEQ_LEN, 1, 0)
            j = r - bi * SEQ_LEN
            bi_v = zero16 + bi
            j_v = zero16 + j
            v0 = rows_v[pb, r, pl.ds(0, 16)]
            v1 = rows_v[pb, r, pl.ds(16, 16)]
            plsc.store_scatter(trans_v.at[pb], [bi_v, iot, j_v], v0)
            plsc.store_scatter(trans_v.at[pb], [bi_v, iot16, j_v], v1)

    start_gather(0, 0)
    start_gather(1, 1)
    for k in range(2):
        wait_gather(k)
        transpose(k)
        start_gather(k + 2, k)
        start_write(k, k)

    def steady(p, carry):
        for b in range(2):
            k = 2 * p + b
            wait_gather(b)
            wait_write(b)
            transpose(b)
            start_gather(k + 2, b)
            start_write(k, b)
        return carry

    lax.fori_loop(1, _NCHUNK // 2 - 1, steady, 0)

    for b in range(2):
        k = _NCHUNK - 2 + b
        wait_gather(b)
        wait_write(b)
        transpose(b)
        start_write(k, b)
    wait_write(0)
    wait_write(1)


@jax.jit
def _run(xq, weight):
    mesh = plsc.VectorSubcoreMesh(core_axis_name="c", subcore_axis_name="s")
    return pl.kernel(
        _body,
        out_type=jax.ShapeDtypeStruct((BATCH, EMBEDDING_DIM, SEQ_LEN),
                                      jnp.float32),
        mesh=mesh,
        scratch_types=[
            pltpu.VMEM((_JT, 8, 128), jnp.int32),
            pltpu.VMEM((_PER_W,), jnp.int32),
            pltpu.VMEM((2, _CHUNK, EMBEDDING_DIM), jnp.float32),
            pltpu.VMEM((2, _BPC, EMBEDDING_DIM, SEQ_LEN), jnp.float32),
            pltpu.SemaphoreType.DMA((2,)),
            pltpu.SemaphoreType.DMA((2,)),
        ],
        compiler_params=pltpu.CompilerParams(
            use_tc_tiling_on_sc=False, needs_layout_passes=False
        ),
    )(xq, weight)


def kernel(x, weight):
    xq = x.T.reshape(_JT, 8, _NW, _BPW).transpose(0, 2, 1, 3)
    out3 = _run(xq, weight)
    return out3.transpose(0, 2, 1)
